# baseline (speedup 1.0000x reference)
import functools

import jax
import jax.numpy as jnp
from jax import lax
from jax.experimental import pallas as pl
from jax.experimental.pallas import tpu as pltpu
from jax.experimental.pallas import tpu_sc as plsc

_N = 10000
_E = 320000
_EA = _E + _N
_EAP = 334080
_D = 128
_L = 5
_NC = 2
_NS = 16
_NW = _NC * _NS
_CH = 128
_TW = 320
_ACCR = 328
_DUMP = 320

_mesh = plsc.VectorSubcoreMesh(core_axis_name="c", subcore_axis_name="s")


@functools.partial(
    pl.kernel,
    out_type=jax.ShapeDtypeStruct((_NW * _TW, _D), jnp.float32),
    mesh=_mesh,
    scratch_types=[
        pltpu.VMEM((48,), jnp.int32),
        pltpu.VMEM((48,), jnp.int32),
        pltpu.VMEM((_CH,), jnp.int32),
        pltpu.VMEM((_CH + 16,), jnp.int32),
        pltpu.VMEM((_CH,), jnp.int32),
        pltpu.VMEM((_CH, _D), jnp.float32),
        pltpu.VMEM((_CH, _D), jnp.float32),
        pltpu.VMEM((_ACCR, _D), jnp.float32),
        pltpu.SemaphoreType.DMA,
        pltpu.SemaphoreType.DMA,
    ],
)
def _seg_fold(h_hbm, etab_hbm, src_hbm, dstloc_hbm, combo_hbm, aoff_hbm,
              anch_hbm, z_hbm, out_hbm, aoff_v, anch_v, src_i, dst_i, cmb_i,
              rows, erows, acc, sem1, sem2):
    c = lax.axis_index("c")
    s = lax.axis_index("s")
    wid = c * _NS + s
    pltpu.sync_copy(z_hbm, acc)
    pltpu.sync_copy(aoff_hbm, aoff_v)
    pltpu.sync_copy(anch_hbm, anch_v)
    o0 = aoff_v[pl.ds(wid, 16)][0]
    nch = anch_v[pl.ds(wid, 16)][0]

    def chunk(j, carry):
        cb = pl.multiple_of(o0 + j * _CH, _CH)
        pltpu.sync_copy(src_hbm.at[pl.ds(cb, _CH)], src_i)
        pltpu.sync_copy(dstloc_hbm.at[pl.ds(cb, _CH)], dst_i.at[pl.ds(0, _CH)])
        pltpu.sync_copy(combo_hbm.at[pl.ds(cb, _CH)], cmb_i)
        g1 = pltpu.async_copy(h_hbm.at[src_i], rows, sem1)
        g2 = pltpu.async_copy(etab_hbm.at[cmb_i], erows, sem2)
        g1.wait()
        g2.wait()

        def edge(e, carry2):
            d = dst_i[pl.ds(e, 16)][0]
            for k in range(8):
                sl = pl.ds(k * 16, 16)
                acc[d, sl] = acc[d, sl] + (rows[e, sl] + erows[e, sl])
            return carry2

        lax.fori_loop(0, _CH, edge, 0, unroll=4)
        return carry

    lax.fori_loop(0, nch, chunk, 0, unroll=False)
    ob = pl.multiple_of(wid * _TW, 8)
    pltpu.sync_copy(acc.at[pl.ds(0, _TW)], out_hbm.at[pl.ds(ob, _TW)])


def _h0_body(x_ref, e1_ref, e2_ref, out_ref):
    x = x_ref[...]
    x0 = x[:, 0:1]
    x1 = x[:, 1:2]
    acc = jnp.zeros((_N, _D), jnp.float32)
    for k in range(3):
        acc = acc + (x0 == k).astype(jnp.float32) * e1_ref[k:k + 1, :]
        acc = acc + (x1 == k).astype(jnp.float32) * e2_ref[k:k + 1, :]
    out_ref[...] = acc


def _h0_call(x, x_emb1, x_emb2):
    return pl.pallas_call(
        _h0_body,
        out_shape=jax.ShapeDtypeStruct((_N, _D), jnp.float32),
    )(x, x_emb1, x_emb2)


def _etab_body(e1_ref, e2_ref, out_ref):
    e1 = e1_ref[...]
    e2 = e2_ref[...]
    for l in range(_L):
        rows = [e1[l, cb // 3:cb // 3 + 1, :] + e2[l, cb % 3:cb % 3 + 1, :]
                for cb in range(9)]
        rows.append(e1[l, 4:5, :] + e2[l, 0:1, :])
        rows.append(jnp.zeros((6, _D), jnp.float32))
        out_ref[l * 16:(l + 1) * 16, :] = jnp.concatenate(rows, axis=0)


def _etab_call(edge_emb1, edge_emb2):
    return pl.pallas_call(
        _etab_body,
        out_shape=jax.ShapeDtypeStruct((_L * 16, _D), jnp.float32),
    )(edge_emb1, edge_emb2)


def _layer_body(last, g_ref, w1_ref, b1_ref, w2_ref, b2_ref, gm_ref, bt_ref,
                out_ref):
    a = g_ref[: _N, :]
    hid = jnp.maximum(
        jnp.dot(a, w1_ref[...], preferred_element_type=jnp.float32)
        + b1_ref[...], 0.0)
    hp = (jnp.dot(hid, w2_ref[...], preferred_element_type=jnp.float32)
          + b2_ref[...])
    mu = jnp.mean(hp, axis=0, keepdims=True)
    var = jnp.mean((hp - mu) ** 2, axis=0, keepdims=True)
    h = (hp - mu) * lax.rsqrt(var + 1e-5) * gm_ref[...] + bt_ref[...]
    if not last:
        h = jnp.maximum(h, 0.0)
    out_ref[...] = h


def _layer_call(last, g, w1, b1, w2, b2, gm, bt):
    return pl.pallas_call(
        functools.partial(_layer_body, last),
        out_shape=jax.ShapeDtypeStruct((_N, _D), jnp.float32),
    )(g, w1, b1, w2, b2, gm, bt)


def kernel(x, edge_index, edge_attr, x_emb1, x_emb2, edge_emb1, edge_emb2,
           W1, b1, W2, b2, gamma, beta):
    x = x.astype(jnp.int32)
    ei = edge_index.astype(jnp.int32)
    ea = edge_attr.astype(jnp.int32)

    loop = jnp.arange(_N, dtype=jnp.int32)
    src_full = jnp.concatenate([ei[0], loop])
    dst_full = jnp.concatenate([ei[1], loop])
    combo_full = jnp.concatenate([ea[:, 0] * 3 + ea[:, 1],
                                  jnp.full((_N,), 9, jnp.int32)])
    tile_id = dst_full // _TW
    perm = jnp.argsort(tile_id, stable=True)
    tid_s = tile_id[perm]
    off = jnp.searchsorted(tid_s, jnp.arange(33, dtype=jnp.int32),
                           side="left").astype(jnp.int32)
    cnt = off[1:] - off[:-1]
    nch = (cnt + _CH - 1) // _CH
    aoff = jnp.concatenate([jnp.zeros((1,), jnp.int32),
                            jnp.cumsum(nch).astype(jnp.int32)]) * _CH
    pos = aoff[tid_s] + jnp.arange(_EA, dtype=jnp.int32) - off[tid_s]
    src_s = jnp.zeros((_EAP,), jnp.int32).at[pos].set(src_full[perm])
    dstloc_s = jnp.full((_EAP,), _DUMP, jnp.int32).at[pos].set(
        dst_full[perm] - tid_s * _TW)
    combo_s = jnp.full((_EAP,), 15, jnp.int32).at[pos].set(combo_full[perm])
    aoff_p = jnp.concatenate([aoff[:32], jnp.zeros((16,), jnp.int32)])
    anch_p = jnp.concatenate([nch, jnp.zeros((16,), jnp.int32)])
    zt = jnp.zeros((_ACCR, _D), jnp.float32)

    h = _h0_call(x, x_emb1, x_emb2)
    etabs = _etab_call(edge_emb1, edge_emb2)
    for l in range(_L):
        g = _seg_fold(h, etabs[l * 16:(l + 1) * 16], src_s, dstloc_s, combo_s,
                      aoff_p, anch_p, zt)
        h = _layer_call(
            l == _L - 1, g, W1[l], b1[l].reshape(1, -1),
            W2[l], b2[l].reshape(1, -1), gamma[l].reshape(1, -1),
            beta[l].reshape(1, -1))
    return h


# speedup vs baseline: 1.0007x; 1.0007x over previous
import functools

import jax
import jax.numpy as jnp
from jax import lax
from jax.experimental import pallas as pl
from jax.experimental.pallas import tpu as pltpu
from jax.experimental.pallas import tpu_sc as plsc

_N = 10000
_E = 320000
_EA = _E + _N
_EAP = 334080
_D = 128
_L = 5
_NC = 2
_NS = 16
_NW = _NC * _NS
_CH = 128
_TW = 320
_ACCR = 328
_DUMP = 320

_mesh = plsc.VectorSubcoreMesh(core_axis_name="c", subcore_axis_name="s")


@functools.partial(
    pl.kernel,
    out_type=jax.ShapeDtypeStruct((_NW * _TW, _D), jnp.float32),
    mesh=_mesh,
    scratch_types=[
        pltpu.VMEM((48,), jnp.int32),
        pltpu.VMEM((48,), jnp.int32),
        pltpu.VMEM((_CH,), jnp.int32),
        pltpu.VMEM((_CH + 16,), jnp.int32),
        pltpu.VMEM((_CH,), jnp.int32),
        pltpu.VMEM((_CH, _D), jnp.float32),
        pltpu.VMEM((_CH, _D), jnp.float32),
        pltpu.VMEM((_ACCR, _D), jnp.float32),
        pltpu.SemaphoreType.DMA,
        pltpu.SemaphoreType.DMA,
    ],
)
def _seg_fold(h_hbm, etab_hbm, src_hbm, dstloc_hbm, combo_hbm, aoff_hbm,
              anch_hbm, z_hbm, out_hbm, aoff_v, anch_v, src_i, dst_i, cmb_i,
              rows, erows, acc, sem1, sem2):
    c = lax.axis_index("c")
    s = lax.axis_index("s")
    wid = c * _NS + s
    pltpu.sync_copy(z_hbm, acc)
    pltpu.sync_copy(aoff_hbm, aoff_v)
    pltpu.sync_copy(anch_hbm, anch_v)
    o0 = aoff_v[pl.ds(wid, 16)][0]
    nch = anch_v[pl.ds(wid, 16)][0]

    def chunk(j, carry):
        cb = pl.multiple_of(o0 + j * _CH, _CH)
        pltpu.sync_copy(src_hbm.at[pl.ds(cb, _CH)], src_i)
        pltpu.sync_copy(dstloc_hbm.at[pl.ds(cb, _CH)], dst_i.at[pl.ds(0, _CH)])
        pltpu.sync_copy(combo_hbm.at[pl.ds(cb, _CH)], cmb_i)
        g1 = pltpu.async_copy(h_hbm.at[src_i], rows, sem1)
        g2 = pltpu.async_copy(etab_hbm.at[cmb_i], erows, sem2)
        g1.wait()
        g2.wait()

        def edge(e, carry2):
            d = dst_i[pl.ds(e, 16)][0]
            for k in range(8):
                sl = pl.ds(k * 16, 16)
                acc[d, sl] = acc[d, sl] + (rows[e, sl] + erows[e, sl])
            return carry2

        lax.fori_loop(0, _CH, edge, 0, unroll=False)
        return carry

    lax.fori_loop(0, nch, chunk, 0, unroll=False)
    ob = pl.multiple_of(wid * _TW, 8)
    pltpu.sync_copy(acc.at[pl.ds(0, _TW)], out_hbm.at[pl.ds(ob, _TW)])


def _h0_body(x_ref, e1_ref, e2_ref, out_ref):
    x = x_ref[...]
    x0 = x[:, 0:1]
    x1 = x[:, 1:2]
    acc = jnp.zeros((_N, _D), jnp.float32)
    for k in range(3):
        acc = acc + (x0 == k).astype(jnp.float32) * e1_ref[k:k + 1, :]
        acc = acc + (x1 == k).astype(jnp.float32) * e2_ref[k:k + 1, :]
    out_ref[...] = acc


def _h0_call(x, x_emb1, x_emb2):
    return pl.pallas_call(
        _h0_body,
        out_shape=jax.ShapeDtypeStruct((_N, _D), jnp.float32),
    )(x, x_emb1, x_emb2)


def _etab_body(e1_ref, e2_ref, out_ref):
    e1 = e1_ref[...]
    e2 = e2_ref[...]
    for l in range(_L):
        rows = [e1[l, cb // 3:cb // 3 + 1, :] + e2[l, cb % 3:cb % 3 + 1, :]
                for cb in range(9)]
        rows.append(e1[l, 4:5, :] + e2[l, 0:1, :])
        rows.append(jnp.zeros((6, _D), jnp.float32))
        out_ref[l * 16:(l + 1) * 16, :] = jnp.concatenate(rows, axis=0)


def _etab_call(edge_emb1, edge_emb2):
    return pl.pallas_call(
        _etab_body,
        out_shape=jax.ShapeDtypeStruct((_L * 16, _D), jnp.float32),
    )(edge_emb1, edge_emb2)


def _layer_body(last, g_ref, w1_ref, b1_ref, w2_ref, b2_ref, gm_ref, bt_ref,
                out_ref):
    a = g_ref[: _N, :]
    hid = jnp.maximum(
        jnp.dot(a, w1_ref[...], preferred_element_type=jnp.float32)
        + b1_ref[...], 0.0)
    hp = (jnp.dot(hid, w2_ref[...], preferred_element_type=jnp.float32)
          + b2_ref[...])
    mu = jnp.mean(hp, axis=0, keepdims=True)
    var = jnp.mean((hp - mu) ** 2, axis=0, keepdims=True)
    h = (hp - mu) * lax.rsqrt(var + 1e-5) * gm_ref[...] + bt_ref[...]
    if not last:
        h = jnp.maximum(h, 0.0)
    out_ref[...] = h


def _layer_call(last, g, w1, b1, w2, b2, gm, bt):
    return pl.pallas_call(
        functools.partial(_layer_body, last),
        out_shape=jax.ShapeDtypeStruct((_N, _D), jnp.float32),
    )(g, w1, b1, w2, b2, gm, bt)


def kernel(x, edge_index, edge_attr, x_emb1, x_emb2, edge_emb1, edge_emb2,
           W1, b1, W2, b2, gamma, beta):
    x = x.astype(jnp.int32)
    ei = edge_index.astype(jnp.int32)
    ea = edge_attr.astype(jnp.int32)

    loop = jnp.arange(_N, dtype=jnp.int32)
    src_full = jnp.concatenate([ei[0], loop])
    dst_full = jnp.concatenate([ei[1], loop])
    combo_full = jnp.concatenate([ea[:, 0] * 3 + ea[:, 1],
                                  jnp.full((_N,), 9, jnp.int32)])
    tile_id = dst_full // _TW
    perm = jnp.argsort(tile_id, stable=True)
    tid_s = tile_id[perm]
    off = jnp.searchsorted(tid_s, jnp.arange(33, dtype=jnp.int32),
                           side="left").astype(jnp.int32)
    cnt = off[1:] - off[:-1]
    nch = (cnt + _CH - 1) // _CH
    aoff = jnp.concatenate([jnp.zeros((1,), jnp.int32),
                            jnp.cumsum(nch).astype(jnp.int32)]) * _CH
    pos = aoff[tid_s] + jnp.arange(_EA, dtype=jnp.int32) - off[tid_s]
    src_s = jnp.zeros((_EAP,), jnp.int32).at[pos].set(src_full[perm])
    dstloc_s = jnp.full((_EAP,), _DUMP, jnp.int32).at[pos].set(
        dst_full[perm] - tid_s * _TW)
    combo_s = jnp.full((_EAP,), 15, jnp.int32).at[pos].set(combo_full[perm])
    aoff_p = jnp.concatenate([aoff[:32], jnp.zeros((16,), jnp.int32)])
    anch_p = jnp.concatenate([nch, jnp.zeros((16,), jnp.int32)])
    zt = jnp.zeros((_ACCR, _D), jnp.float32)

    h = _h0_call(x, x_emb1, x_emb2)
    etabs = _etab_call(edge_emb1, edge_emb2)
    for l in range(_L):
        g = _seg_fold(h, etabs[l * 16:(l + 1) * 16], src_s, dstloc_s, combo_s,
                      aoff_p, anch_p, zt)
        h = _layer_call(
            l == _L - 1, g, W1[l], b1[l].reshape(1, -1),
            W2[l], b2[l].reshape(1, -1), gamma[l].reshape(1, -1),
            beta[l].reshape(1, -1))
    return h
